# baseline (speedup 1.0000x reference)
import functools

import jax
import jax.numpy as jnp
from jax import lax
from jax.experimental import pallas as pl
from jax.experimental.pallas import tpu as pltpu
from jax.experimental.pallas import tpu_sc as plsc

_N = 10000
_D = 256
_H = 128
_NB = 10
_NC, _NS = 2, 16
_K = 112
_EPAD = 172032
_NACC = 10112
_RPT = _NACC // _NS
_CH_FULL = _EPAD // (_NS * _K)
_EHALF = _EPAD // 2
_CH_HALF = _EHALF // (_NS * _K)
_BIDX = 1024
_BPW = _BIDX // (_NC * _NS)

_mesh = plsc.VectorSubcoreMesh(
    core_axis_name="c", subcore_axis_name="s", num_cores=_NC, num_subcores=_NS)


def _spmm_body(y0, y1, srcp, dstp, zrows, out0, out1,
               src_bufs, dst_bufs, row_bufs, acc, gsems, ssems):
  c = lax.axis_index("c")
  s = lax.axis_index("s")

  def half(y, out):
    tb = s * _CH_FULL * _K

    def ld_src(j, b):
      pltpu.sync_copy(srcp.at[pl.ds(tb + j * _K, _K)], src_bufs[b])

    def ld_dst(j, b):
      pltpu.sync_copy(dstp.at[pl.ds(tb + j * _K, _K)], dst_bufs[b])

    def fire_gather(b):
      pltpu.async_copy(y.at[src_bufs[b]], row_bufs[b], gsems[b])

    def wait_gather(b):
      pltpu.make_async_copy(y.at[src_bufs[b]], row_bufs[b], gsems[b]).wait()

    def fire_scatter(b):
      return pltpu.async_copy(row_bufs[b], acc.at[dst_bufs[b]], ssems[b],
                              add=True)

    pltpu.sync_copy(zrows.at[pl.ds(s * _RPT, _RPT)],
                    acc.at[pl.ds(s * _RPT, _RPT)])
    plsc.subcore_barrier()

    for b in range(3):
      ld_src(b, b)
      fire_gather(b)

    def triple(i, carry):
      j = 3 * i
      descs = []
      for b in range(3):
        wait_gather(b)
        ld_dst(j + b, b)
        descs.append(fire_scatter(b))
      for b in range(3):
        descs[b].wait()

        @pl.when(j + 3 + b < _CH_FULL)
        def _():
          ld_src(j + 3 + b, b)
          fire_gather(b)
      return carry

    lax.fori_loop(0, _CH_FULL // 3, triple, 0)
    plsc.subcore_barrier()
    pltpu.sync_copy(acc.at[pl.ds(s * _RPT, _RPT)],
                    out.at[pl.ds(s * _RPT, _RPT)])

  @pl.when(c == 0)
  def _():
    half(y0, out0)

  @pl.when(c == 1)
  def _():
    half(y1, out1)


_spmm = functools.partial(
    pl.kernel, _spmm_body,
    out_type=[jax.ShapeDtypeStruct((_NACC, _H), jnp.float32),
              jax.ShapeDtypeStruct((_NACC, _H), jnp.float32)],
    mesh=_mesh,
    scratch_types=[
        [pltpu.VMEM((_K,), jnp.int32)] * 3,
        [pltpu.VMEM((_K,), jnp.int32)] * 3,
        [pltpu.VMEM((_K, _H), jnp.float32)] * 3,
        pltpu.VMEM_SHARED((_NACC, _H), jnp.float32),
        [pltpu.SemaphoreType.DMA] * 3,
        [pltpu.SemaphoreType.DMA] * 3,
    ])()


def _deg_body(dstp, zrows16, ones16, d0, d1,
              dst_v, ones_v, acc, sem):
  c = lax.axis_index("c")
  s = lax.axis_index("s")
  pltpu.sync_copy(zrows16.at[pl.ds(s * _RPT, _RPT)],
                  acc.at[pl.ds(s * _RPT, _RPT)])
  pltpu.sync_copy(ones16, ones_v)
  plsc.subcore_barrier()

  def chunk(i, carry):
    base = c * _EHALF + (s * _CH_HALF + i) * _K
    pltpu.sync_copy(dstp.at[pl.ds(base, _K)], dst_v)
    pltpu.sync_copy(ones_v, acc.at[dst_v], add=True)
    return carry

  lax.fori_loop(0, _CH_HALF, chunk, 0)
  plsc.subcore_barrier()

  def half(out):
    pltpu.sync_copy(acc.at[pl.ds(s * _RPT, _RPT)],
                    out.at[pl.ds(s * _RPT, _RPT)])

  @pl.when(c == 0)
  def _():
    half(d0)

  @pl.when(c == 1)
  def _():
    half(d1)


_deg = functools.partial(
    pl.kernel, _deg_body,
    out_type=[jax.ShapeDtypeStruct((_NACC, 16), jnp.float32),
              jax.ShapeDtypeStruct((_NACC, 16), jnp.float32)],
    mesh=_mesh,
    scratch_types=[
        pltpu.VMEM((_K,), jnp.int32),
        pltpu.VMEM((_K, 16), jnp.float32),
        pltpu.VMEM_SHARED((_NACC, 16), jnp.float32),
        pltpu.SemaphoreType.DMA,
    ])()


def _gather_body(emb, idxp, raw, idx_v, rows_v, sem):
  c = lax.axis_index("c")
  s = lax.axis_index("s")
  wid = s * _NC + c
  base = wid * _BPW
  pltpu.sync_copy(idxp.at[pl.ds(base, _BPW)], idx_v)
  pltpu.async_copy(emb.at[idx_v], rows_v, sem).wait()
  pltpu.sync_copy(rows_v, raw.at[pl.ds(base, _BPW)])


_gather_rows = functools.partial(
    pl.kernel, _gather_body,
    out_type=jax.ShapeDtypeStruct((_BIDX, _D), jnp.float32),
    mesh=_mesh,
    scratch_types=[
        pltpu.VMEM((_BPW,), jnp.int32),
        pltpu.VMEM((_BPW, _D), jnp.float32),
        pltpu.SemaphoreType.DMA,
    ])()


_R = 1000
_DOT = functools.partial(jnp.dot, preferred_element_type=jnp.float32,
                         precision=jax.lax.Precision.HIGHEST)


def _dinv(d0_ref, d1_ref):
  deg = d0_ref[...][:, :1] + d1_ref[...][:, :1]
  return jnp.where(deg > 0, 1.0 / jnp.sqrt(deg), 0.0)


def _row_spec(cols):
  return pl.BlockSpec((_R, cols), lambda i: (i, 0))


def _full_spec(rows, cols):
  return pl.BlockSpec((rows, cols), lambda i: (0, 0))


def _stage_call(body, n_out_halves, extra_outs, ins, in_specs):
  out_shape = ([jax.ShapeDtypeStruct((_N, _D), jnp.float32)] * extra_outs
               + [jax.ShapeDtypeStruct((_N, _H), jnp.float32)] * n_out_halves)
  out_specs = ([_row_spec(_D)] * extra_outs + [_row_spec(_H)] * n_out_halves)
  return pl.pallas_call(
      body, grid=(_N // _R,), out_shape=out_shape,
      in_specs=in_specs, out_specs=out_specs)(*ins)


def _t1_body(x_ref, d0_ref, d1_ref, w_ref, u0_ref, u1_ref):
  di = _dinv(d0_ref, d1_ref)
  u = _DOT(x_ref[...], w_ref[...]) * di
  u0_ref[...] = u[:, :_H]
  u1_ref[...] = u[:, _H:]


def _t1(x, d0, d1, W):
  return _stage_call(
      _t1_body, 2, 0, (x, d0, d1, W),
      [_row_spec(_D), _row_spec(16), _row_spec(16), _full_spec(_D, _D)])


def _conv_body(act, emit_pre, s0_ref, s1_ref, d0_ref, d1_ref, b_ref, w_ref,
               *out_refs):
  di = _dinv(d0_ref, d1_ref)
  pre = di * jnp.concatenate([s0_ref[...], s1_ref[...]], axis=1) + b_ref[...]
  if act == "relu":
    pre = jnp.maximum(pre, 0.0)
  u = _DOT(pre, w_ref[...]) * di
  if emit_pre:
    out_refs[0][...] = pre
  out_refs[-2][...] = u[:, :_H]
  out_refs[-1][...] = u[:, _H:]


def _conv_stage(s0, s1, d0, d1, b, W, act, emit_pre):
  body = functools.partial(_conv_body, act, emit_pre)
  return _stage_call(
      body, 2, 1 if emit_pre else 0, (s0, s1, d0, d1, b, W),
      [_row_spec(_H), _row_spec(_H), _row_spec(16), _row_spec(16),
       _full_spec(1, _D), _full_spec(_D, _D)])


def _elu(h):
  return jnp.where(h > 0, h, jnp.exp(h) - 1.0)


def _tail_body(s0_ref, s1_ref, d0_ref, d1_ref, b_ref, e1_ref, w2_ref,
               wi_ref, bi_ref, wh_ref, bh_ref, wo_ref, bo_ref, ox_ref,
               w0_ref, u0_ref, u1_ref):
  di = _dinv(d0_ref, d1_ref)
  e2 = (di * jnp.concatenate([s0_ref[...], s1_ref[...]], axis=1)
        + b_ref[...] + e1_ref[...])
  embed = w2_ref[0, 0] * e2
  h = _elu(_DOT(embed, wi_ref[...]) + bi_ref[...])
  h = _elu(_DOT(h, wh_ref[...]) + bh_ref[...])
  xn = (_DOT(h, wo_ref[...]) + bo_ref[...]) * ox_ref[...]
  u = _DOT(xn, w0_ref[...]) * di
  u0_ref[...] = u[:, :_H]
  u1_ref[...] = u[:, _H:]


def _think_tail(s0, s1, d0, d1, b, e1, w2, Wi, bi, Wh, bh, Wo, bo,
                origin_x, W0):
  return _stage_call(
      _tail_body, 2, 0,
      (s0, s1, d0, d1, b, e1, w2, Wi, bi, Wh, bh, Wo, bo, origin_x, W0),
      [_row_spec(_H), _row_spec(_H), _row_spec(16), _row_spec(16),
       _full_spec(1, _D), _row_spec(_D), _full_spec(1, 1),
       _full_spec(_D, _D), _full_spec(1, _D), _full_spec(_D, _D),
       _full_spec(1, _D), _full_spec(_D, _D), _full_spec(1, _D),
       _row_spec(_D), _full_spec(_D, _D)])


def _f4_body(s0_ref, s1_ref, d0_ref, d1_ref, b_ref, wa_ref, ba_ref, pl_ref,
             o_ref):
  di = _dinv(d0_ref, d1_ref)
  embed = (di * jnp.concatenate([s0_ref[...], s1_ref[...]], axis=1)
           + b_ref[...])
  score = _DOT(embed, wa_ref[...]) + ba_ref[...]
  weight = jax.nn.softmax(score, axis=1)
  o_ref[...] = embed + _DOT(weight, pl_ref[...])


def _f4(s0, s1, d0, d1, b, Wa, ba, p_list):
  return pl.pallas_call(
      _f4_body, grid=(_N // _R,),
      out_shape=jax.ShapeDtypeStruct((_N, _D), jnp.float32),
      in_specs=[_row_spec(_H), _row_spec(_H), _row_spec(16), _row_spec(16),
                _full_spec(1, _D), _full_spec(_D, 5), _full_spec(1, 5),
                _full_spec(5, _D)],
      out_specs=_row_spec(_D))(s0, s1, d0, d1, b, Wa, ba, p_list)


def _head_body(raw_ref, lab_ref, train_ref, o_ref):
  raw = raw_ref[...]
  onehot = (lab_ref[...] == lax.broadcasted_iota(jnp.int32, (1, _NB), 1)
            ).astype(jnp.float32)
  sums = lax.dot_general(onehot, raw, (((0,), (0,)), ((), ())),
                         preferred_element_type=jnp.float32,
                         precision=jax.lax.Precision.HIGHEST)
  ones_col = jnp.ones((raw.shape[0], 1), jnp.float32)
  cnts = lax.dot_general(onehot, ones_col, (((0,), (0,)), ((), ())),
                         preferred_element_type=jnp.float32,
                         precision=jax.lax.Precision.HIGHEST)
  ave = sums / jnp.clip(cnts, 1.0)
  ave = jnp.where(train_ref[0, 0] == 1.0, ave, jnp.zeros_like(ave))
  rn = jnp.sqrt(jnp.sum(raw * raw, axis=1, keepdims=True))
  an = jnp.sqrt(jnp.sum(ave * ave, axis=1, keepdims=True))
  num = lax.dot_general(raw, ave, (((1,), (1,)), ((), ())),
                        preferred_element_type=jnp.float32,
                        precision=jax.lax.Precision.HIGHEST)
  denom = jnp.clip(lax.dot_general(rn, an, (((1,), (1,)), ((), ())),
                                   preferred_element_type=jnp.float32,
                                   precision=jax.lax.Precision.HIGHEST),
                   1e-8)
  o_ref[...] = jax.nn.softmax(num / denom, axis=1)


def _head(raw, lab2d, train_s):
  nidx = raw.shape[0]
  return pl.pallas_call(
      _head_body,
      out_shape=jax.ShapeDtypeStruct((nidx, _NB), jnp.float32))(
          raw, lab2d, train_s)


def kernel(x, params, edge_index, idx, labels, train):
  n = x.shape[0]
  loop = jnp.arange(n, dtype=jnp.int32)
  src = jnp.concatenate([edge_index[0].astype(jnp.int32), loop])
  dst = jnp.concatenate([edge_index[1].astype(jnp.int32), loop])
  epad = _EPAD - src.shape[0]
  srcp = jnp.concatenate([src, jnp.zeros((epad,), jnp.int32)])
  dstp = jnp.concatenate([dst, jnp.full((epad,), n, jnp.int32)])

  zrows = jnp.zeros((_NACC, _H), jnp.float32)
  zrows16 = jnp.zeros((_NACC, 16), jnp.float32)
  ones16 = jnp.ones((_K, 16), jnp.float32)

  d0, d1 = _deg(dstp, zrows16, ones16)

  Wg, bg = params["gcn_W"], params["gcn_b"]
  bgr = [b.reshape(1, _D) for b in bg]
  w2 = jnp.asarray(params["gcn_weight2"], jnp.float32).reshape(1, 1)

  origin_x = x
  u0, u1 = _t1(x, d0, d1, Wg[0])
  for layer in params["cond"]:
    s0, s1 = _spmm(u0, u1, srcp, dstp, zrows)
    e1, v0, v1 = _conv_stage(s0, s1, d0, d1, bgr[0], Wg[1],
                             act=None, emit_pre=True)
    s0, s1 = _spmm(v0, v1, srcp, dstp, zrows)
    u0, u1 = _think_tail(s0, s1, d0, d1, bgr[1], e1, w2,
                         layer["Wi"], layer["bi"].reshape(1, _D),
                         layer["Wh"], layer["bh"].reshape(1, _D),
                         layer["Wo"], layer["bo"].reshape(1, _D),
                         origin_x, Wg[0])

  s0, s1 = _spmm(u0, u1, srcp, dstp, zrows)
  v0, v1 = _conv_stage(s0, s1, d0, d1, bgr[0], Wg[1], act="relu",
                       emit_pre=False)
  s0, s1 = _spmm(v0, v1, srcp, dstp, zrows)
  v0, v1 = _conv_stage(s0, s1, d0, d1, bgr[1], Wg[2], act="relu",
                       emit_pre=False)
  s0, s1 = _spmm(v0, v1, srcp, dstp, zrows)
  embed2 = _f4(s0, s1, d0, d1, bg[2].reshape(1, _D),
               params["Wa"], params["ba"].reshape(1, 5), params["p_list"])

  nidx = idx.shape[0]
  idxp = jnp.concatenate(
      [idx.astype(jnp.int32),
       jnp.zeros((_BIDX - nidx,), jnp.int32)])
  rawp = _gather_rows(embed2, idxp)
  raw = rawp[:nidx]

  lab2d = labels.astype(jnp.int32).reshape(nidx, 1)
  train_s = jnp.asarray(train, jnp.float32).reshape(1, 1)
  return _head(raw, lab2d, train_s)


# speedup vs baseline: 1.2330x; 1.2330x over previous
import functools

import jax
import jax.numpy as jnp
from jax import lax
from jax.experimental import pallas as pl
from jax.experimental.pallas import tpu as pltpu
from jax.experimental.pallas import tpu_sc as plsc

_N = 10000
_D = 256
_H = 128
_NB = 10
_NC, _NS = 2, 16
_K = 112
_EPAD = 161280
_NACC = 10112
_RPT = _NACC // _NS
_CH_FULL = _EPAD // (_NS * _K)
_EHALF = _EPAD // 2
_CH_HALF = _EHALF // (_NS * _K)
_BIDX = 1024
_BPW = _BIDX // (_NC * _NS)

_mesh = plsc.VectorSubcoreMesh(
    core_axis_name="c", subcore_axis_name="s", num_cores=_NC, num_subcores=_NS)


def _spmm_body(y0, y1, srcp, dstp, zrows, out0, out1,
               src_bufs, dst_bufs, row_bufs, acc, gsems, ssems):
  c = lax.axis_index("c")
  s = lax.axis_index("s")

  def half(y, out):
    tb = s * _CH_FULL * _K

    def ld_src(j, b):
      pltpu.sync_copy(srcp.at[pl.ds(tb + j * _K, _K)], src_bufs[b])

    def ld_dst(j, b):
      pltpu.sync_copy(dstp.at[pl.ds(tb + j * _K, _K)], dst_bufs[b])

    def fire_gather(b):
      pltpu.async_copy(y.at[src_bufs[b]], row_bufs[b], gsems[b])

    def wait_gather(b):
      pltpu.make_async_copy(y.at[src_bufs[b]], row_bufs[b], gsems[b]).wait()

    def fire_scatter(b):
      return pltpu.async_copy(row_bufs[b], acc.at[dst_bufs[b]], ssems[b],
                              add=True)

    pltpu.sync_copy(zrows.at[pl.ds(s * _RPT, _RPT)],
                    acc.at[pl.ds(s * _RPT, _RPT)])
    plsc.subcore_barrier()

    for b in range(3):
      ld_src(b, b)
      fire_gather(b)

    def triple(i, carry):
      j = 3 * i
      descs = []
      for b in range(3):
        wait_gather(b)
        ld_dst(j + b, b)
        descs.append(fire_scatter(b))
      for b in range(3):
        descs[b].wait()

        @pl.when(j + 3 + b < _CH_FULL)
        def _():
          ld_src(j + 3 + b, b)
          fire_gather(b)
      return carry

    lax.fori_loop(0, _CH_FULL // 3, triple, 0)
    plsc.subcore_barrier()
    pltpu.sync_copy(acc.at[pl.ds(s * _RPT, _RPT)],
                    out.at[pl.ds(s * _RPT, _RPT)])

  @pl.when(c == 0)
  def _():
    half(y0, out0)

  @pl.when(c == 1)
  def _():
    half(y1, out1)


_spmm = functools.partial(
    pl.kernel, _spmm_body,
    out_type=[jax.ShapeDtypeStruct((_NACC, _H), jnp.float32),
              jax.ShapeDtypeStruct((_NACC, _H), jnp.float32)],
    mesh=_mesh,
    scratch_types=[
        [pltpu.VMEM((_K,), jnp.int32)] * 3,
        [pltpu.VMEM((_K,), jnp.int32)] * 3,
        [pltpu.VMEM((_K, _H), jnp.float32)] * 3,
        pltpu.VMEM_SHARED((_NACC, _H), jnp.float32),
        [pltpu.SemaphoreType.DMA] * 3,
        [pltpu.SemaphoreType.DMA] * 3,
    ])()


def _deg_body(dstp, zrows16, ones16, d0, d1,
              dst_v, ones_v, acc, sem):
  c = lax.axis_index("c")
  s = lax.axis_index("s")
  pltpu.sync_copy(zrows16.at[pl.ds(s * _RPT, _RPT)],
                  acc.at[pl.ds(s * _RPT, _RPT)])
  pltpu.sync_copy(ones16, ones_v)
  plsc.subcore_barrier()

  def chunk(i, carry):
    base = c * _EHALF + (s * _CH_HALF + i) * _K
    pltpu.sync_copy(dstp.at[pl.ds(base, _K)], dst_v)
    pltpu.sync_copy(ones_v, acc.at[dst_v], add=True)
    return carry

  lax.fori_loop(0, _CH_HALF, chunk, 0)
  plsc.subcore_barrier()

  def half(out):
    pltpu.sync_copy(acc.at[pl.ds(s * _RPT, _RPT)],
                    out.at[pl.ds(s * _RPT, _RPT)])

  @pl.when(c == 0)
  def _():
    half(d0)

  @pl.when(c == 1)
  def _():
    half(d1)


_deg = functools.partial(
    pl.kernel, _deg_body,
    out_type=[jax.ShapeDtypeStruct((_NACC, 16), jnp.float32),
              jax.ShapeDtypeStruct((_NACC, 16), jnp.float32)],
    mesh=_mesh,
    scratch_types=[
        pltpu.VMEM((_K,), jnp.int32),
        pltpu.VMEM((_K, 16), jnp.float32),
        pltpu.VMEM_SHARED((_NACC, 16), jnp.float32),
        pltpu.SemaphoreType.DMA,
    ])()


def _gather_body(emb, idxp, raw, idx_v, rows_v, sem):
  c = lax.axis_index("c")
  s = lax.axis_index("s")
  wid = s * _NC + c
  base = wid * _BPW
  pltpu.sync_copy(idxp.at[pl.ds(base, _BPW)], idx_v)
  pltpu.async_copy(emb.at[idx_v], rows_v, sem).wait()
  pltpu.sync_copy(rows_v, raw.at[pl.ds(base, _BPW)])


_gather_rows = functools.partial(
    pl.kernel, _gather_body,
    out_type=jax.ShapeDtypeStruct((_BIDX, _D), jnp.float32),
    mesh=_mesh,
    scratch_types=[
        pltpu.VMEM((_BPW,), jnp.int32),
        pltpu.VMEM((_BPW, _D), jnp.float32),
        pltpu.SemaphoreType.DMA,
    ])()


_R = 1000
_DOT = functools.partial(jnp.dot, preferred_element_type=jnp.float32,
                         precision=jax.lax.Precision.HIGHEST)


def _dinv(d0_ref, d1_ref):
  deg = d0_ref[...][:, :1] + d1_ref[...][:, :1] + 1.0
  return 1.0 / jnp.sqrt(deg)


def _row_spec(cols):
  return pl.BlockSpec((_R, cols), lambda i: (i, 0))


def _full_spec(rows, cols):
  return pl.BlockSpec((rows, cols), lambda i: (0, 0))


def _stage_call(body, n_out_halves, extra_outs, ins, in_specs):
  out_shape = ([jax.ShapeDtypeStruct((_N, _D), jnp.float32)] * extra_outs
               + [jax.ShapeDtypeStruct((_N, _H), jnp.float32)] * n_out_halves)
  out_specs = ([_row_spec(_D)] * extra_outs + [_row_spec(_H)] * n_out_halves)
  return pl.pallas_call(
      body, grid=(_N // _R,), out_shape=out_shape,
      in_specs=in_specs, out_specs=out_specs)(*ins)


def _t1_body(x_ref, d0_ref, d1_ref, w_ref, u0_ref, u1_ref):
  di = _dinv(d0_ref, d1_ref)
  u = _DOT(x_ref[...], w_ref[...]) * di
  u0_ref[...] = u[:, :_H]
  u1_ref[...] = u[:, _H:]


def _t1(x, d0, d1, W):
  return _stage_call(
      _t1_body, 2, 0, (x, d0, d1, W),
      [_row_spec(_D), _row_spec(16), _row_spec(16), _full_spec(_D, _D)])


def _conv_body(act, emit_pre, s0_ref, s1_ref, u0_ref, u1_ref,
               d0_ref, d1_ref, b_ref, w_ref, *out_refs):
  di = _dinv(d0_ref, d1_ref)
  pre = di * jnp.concatenate([s0_ref[...] + u0_ref[...],
                              s1_ref[...] + u1_ref[...]], axis=1) + b_ref[...]
  if act == "relu":
    pre = jnp.maximum(pre, 0.0)
  u = _DOT(pre, w_ref[...]) * di
  if emit_pre:
    out_refs[0][...] = pre
  out_refs[-2][...] = u[:, :_H]
  out_refs[-1][...] = u[:, _H:]


def _conv_stage(s0, s1, u0, u1, d0, d1, b, W, act, emit_pre):
  body = functools.partial(_conv_body, act, emit_pre)
  return _stage_call(
      body, 2, 1 if emit_pre else 0, (s0, s1, u0, u1, d0, d1, b, W),
      [_row_spec(_H), _row_spec(_H), _row_spec(_H), _row_spec(_H),
       _row_spec(16), _row_spec(16), _full_spec(1, _D), _full_spec(_D, _D)])


def _elu(h):
  return jnp.where(h > 0, h, jnp.exp(h) - 1.0)


def _tail_body(s0_ref, s1_ref, v0_ref, v1_ref, d0_ref, d1_ref, b_ref,
               e1_ref, w2_ref, wi_ref, bi_ref, wh_ref, bh_ref, wo_ref,
               bo_ref, ox_ref, w0_ref, u0_ref, u1_ref):
  di = _dinv(d0_ref, d1_ref)
  e2 = (di * jnp.concatenate([s0_ref[...] + v0_ref[...],
                              s1_ref[...] + v1_ref[...]], axis=1)
        + b_ref[...] + e1_ref[...])
  embed = w2_ref[0, 0] * e2
  h = _elu(_DOT(embed, wi_ref[...]) + bi_ref[...])
  h = _elu(_DOT(h, wh_ref[...]) + bh_ref[...])
  xn = (_DOT(h, wo_ref[...]) + bo_ref[...]) * ox_ref[...]
  u = _DOT(xn, w0_ref[...]) * di
  u0_ref[...] = u[:, :_H]
  u1_ref[...] = u[:, _H:]


def _think_tail(s0, s1, v0, v1, d0, d1, b, e1, w2, Wi, bi, Wh, bh, Wo, bo,
                origin_x, W0):
  return _stage_call(
      _tail_body, 2, 0,
      (s0, s1, v0, v1, d0, d1, b, e1, w2, Wi, bi, Wh, bh, Wo, bo,
       origin_x, W0),
      [_row_spec(_H), _row_spec(_H), _row_spec(_H), _row_spec(_H),
       _row_spec(16), _row_spec(16), _full_spec(1, _D), _row_spec(_D),
       _full_spec(1, 1), _full_spec(_D, _D), _full_spec(1, _D),
       _full_spec(_D, _D), _full_spec(1, _D), _full_spec(_D, _D),
       _full_spec(1, _D), _row_spec(_D), _full_spec(_D, _D)])


def _f4_body(s0_ref, s1_ref, u0_ref, u1_ref, d0_ref, d1_ref, b_ref,
             wa_ref, ba_ref, pl_ref, o_ref):
  di = _dinv(d0_ref, d1_ref)
  embed = (di * jnp.concatenate([s0_ref[...] + u0_ref[...],
                                 s1_ref[...] + u1_ref[...]], axis=1)
           + b_ref[...])
  score = _DOT(embed, wa_ref[...]) + ba_ref[...]
  weight = jax.nn.softmax(score, axis=1)
  o_ref[...] = embed + _DOT(weight, pl_ref[...])


def _f4(s0, s1, u0, u1, d0, d1, b, Wa, ba, p_list):
  return pl.pallas_call(
      _f4_body, grid=(_N // _R,),
      out_shape=jax.ShapeDtypeStruct((_N, _D), jnp.float32),
      in_specs=[_row_spec(_H), _row_spec(_H), _row_spec(_H), _row_spec(_H),
                _row_spec(16), _row_spec(16), _full_spec(1, _D),
                _full_spec(_D, 5), _full_spec(1, 5), _full_spec(5, _D)],
      out_specs=_row_spec(_D))(s0, s1, u0, u1, d0, d1, b, Wa, ba, p_list)


def _head_body(raw_ref, lab_ref, train_ref, o_ref):
  raw = raw_ref[...]
  onehot = (lab_ref[...] == lax.broadcasted_iota(jnp.int32, (1, _NB), 1)
            ).astype(jnp.float32)
  sums = lax.dot_general(onehot, raw, (((0,), (0,)), ((), ())),
                         preferred_element_type=jnp.float32,
                         precision=jax.lax.Precision.HIGHEST)
  ones_col = jnp.ones((raw.shape[0], 1), jnp.float32)
  cnts = lax.dot_general(onehot, ones_col, (((0,), (0,)), ((), ())),
                         preferred_element_type=jnp.float32,
                         precision=jax.lax.Precision.HIGHEST)
  ave = sums / jnp.clip(cnts, 1.0)
  ave = jnp.where(train_ref[0, 0] == 1.0, ave, jnp.zeros_like(ave))
  rn = jnp.sqrt(jnp.sum(raw * raw, axis=1, keepdims=True))
  an = jnp.sqrt(jnp.sum(ave * ave, axis=1, keepdims=True))
  num = lax.dot_general(raw, ave, (((1,), (1,)), ((), ())),
                        preferred_element_type=jnp.float32,
                        precision=jax.lax.Precision.HIGHEST)
  denom = jnp.clip(lax.dot_general(rn, an, (((1,), (1,)), ((), ())),
                                   preferred_element_type=jnp.float32,
                                   precision=jax.lax.Precision.HIGHEST),
                   1e-8)
  o_ref[...] = jax.nn.softmax(num / denom, axis=1)


def _head(raw, lab2d, train_s):
  nidx = raw.shape[0]
  return pl.pallas_call(
      _head_body,
      out_shape=jax.ShapeDtypeStruct((nidx, _NB), jnp.float32))(
          raw, lab2d, train_s)


def kernel(x, params, edge_index, idx, labels, train):
  n = x.shape[0]
  src = edge_index[0].astype(jnp.int32)
  dst = edge_index[1].astype(jnp.int32)
  epad = _EPAD - src.shape[0]
  srcp = jnp.concatenate([src, jnp.zeros((epad,), jnp.int32)])
  dstp = jnp.concatenate([dst, jnp.full((epad,), n, jnp.int32)])

  zrows = jnp.zeros((_NACC, _H), jnp.float32)
  zrows16 = jnp.zeros((_NACC, 16), jnp.float32)
  ones16 = jnp.ones((_K, 16), jnp.float32)

  d0, d1 = _deg(dstp, zrows16, ones16)

  Wg, bg = params["gcn_W"], params["gcn_b"]
  bgr = [b.reshape(1, _D) for b in bg]
  w2 = jnp.asarray(params["gcn_weight2"], jnp.float32).reshape(1, 1)

  origin_x = x
  u0, u1 = _t1(x, d0, d1, Wg[0])
  for layer in params["cond"]:
    s0, s1 = _spmm(u0, u1, srcp, dstp, zrows)
    e1, v0, v1 = _conv_stage(s0, s1, u0, u1, d0, d1, bgr[0], Wg[1],
                             act=None, emit_pre=True)
    s0, s1 = _spmm(v0, v1, srcp, dstp, zrows)
    u0, u1 = _think_tail(s0, s1, v0, v1, d0, d1, bgr[1], e1, w2,
                         layer["Wi"], layer["bi"].reshape(1, _D),
                         layer["Wh"], layer["bh"].reshape(1, _D),
                         layer["Wo"], layer["bo"].reshape(1, _D),
                         origin_x, Wg[0])

  s0, s1 = _spmm(u0, u1, srcp, dstp, zrows)
  v0, v1 = _conv_stage(s0, s1, u0, u1, d0, d1, bgr[0], Wg[1], act="relu",
                       emit_pre=False)
  s0, s1 = _spmm(v0, v1, srcp, dstp, zrows)
  u0, u1 = v0, v1
  v0, v1 = _conv_stage(s0, s1, u0, u1, d0, d1, bgr[1], Wg[2], act="relu",
                       emit_pre=False)
  s0, s1 = _spmm(v0, v1, srcp, dstp, zrows)
  embed2 = _f4(s0, s1, v0, v1, d0, d1, bg[2].reshape(1, _D),
               params["Wa"], params["ba"].reshape(1, 5), params["p_list"])

  nidx = idx.shape[0]
  idxp = jnp.concatenate(
      [idx.astype(jnp.int32),
       jnp.zeros((_BIDX - nidx,), jnp.int32)])
  rawp = _gather_rows(embed2, idxp)
  raw = rawp[:nidx]

  lab2d = labels.astype(jnp.int32).reshape(nidx, 1)
  train_s = jnp.asarray(train, jnp.float32).reshape(1, 1)
  return _head(raw, lab2d, train_s)
